# Initial kernel scaffold; baseline (speedup 1.0000x reference)
import functools

import jax
import jax.numpy as jnp
from jax import lax
from jax.experimental import pallas as pl
from jax.experimental.pallas import tpu as pltpu
from jax.experimental.pallas import tpu_sc as plsc

N = 10000
E = 320000
D = 128
NLAYERS = 5
NGRAPHS = 128

NC = 2
NS = 16
NW = NC * NS

EPT = E // NW
ECHUNK = 80
NECHUNK = EPT // ECHUNK
NPT = 320
RPS = N // NS

_mesh = plsc.VectorSubcoreMesh(core_axis_name="c", subcore_axis_name="s",
                               num_cores=NC, num_subcores=NS)


def _sc_encode_body(aidx_hbm, row_hbm, atom_hbm, h0_out, cnt_out,
                    idx_v, gbuf, hbuf, rowv, ones_v, zbuf, cnt_sh, sem):
    c = lax.axis_index("c")
    s = lax.axis_index("s")
    wid = c * NS + s

    def _z(i, _):
        zbuf[i, :] = jnp.zeros((16,), jnp.float32)
        return 0
    lax.fori_loop(0, RPS, _z, 0)
    pltpu.sync_copy(zbuf, cnt_sh.at[pl.ds(s * RPS, RPS), :])

    def _o(i, _):
        ones_v[i, :] = jnp.ones((16,), jnp.float32)
        return 0
    lax.fori_loop(0, ECHUNK, _o, 0)
    plsc.subcore_barrier()

    nnodes = jnp.maximum(0, jnp.minimum(NPT, N - wid * NPT))
    nch = nnodes // 8

    def _atom(i, _):
        nbase = wid * NPT + i * 8
        pltpu.sync_copy(aidx_hbm.at[pl.ds(nbase * 9, 72)], idx_v)
        pltpu.async_copy(atom_hbm.at[idx_v], gbuf, sem).wait()
        for j in range(8):
            for d in range(8):
                acc = gbuf[j * 9, pl.ds(d * 16, 16)]
                for k in range(1, 9):
                    acc = acc + gbuf[j * 9 + k, pl.ds(d * 16, 16)]
                hbuf[j, pl.ds(d * 16, 16)] = acc
        pltpu.sync_copy(hbuf, h0_out.at[pl.ds(nbase, 8), :])
        return 0
    lax.fori_loop(0, nch, _atom, 0)

    def _deg(i, _):
        ebase = wid * EPT + i * ECHUNK
        pltpu.sync_copy(row_hbm.at[pl.ds(ebase, ECHUNK)], rowv)
        pltpu.sync_copy(ones_v, cnt_sh.at[rowv], add=True)
        return 0
    lax.fori_loop(0, NECHUNK, _deg, 0)
    plsc.subcore_barrier()

    pltpu.sync_copy(cnt_sh.at[pl.ds(s * RPS, RPS), :], zbuf)
    pltpu.sync_copy(zbuf, cnt_out.at[c, pl.ds(s * RPS, RPS), :])


_sc_encode = pl.kernel(
    _sc_encode_body,
    out_type=(jax.ShapeDtypeStruct((N, D), jnp.float32),
              jax.ShapeDtypeStruct((NC, N, 16), jnp.float32)),
    mesh=_mesh,
    scratch_types=[
        pltpu.VMEM((72,), jnp.int32),
        pltpu.VMEM((72, D), jnp.float32),
        pltpu.VMEM((8, D), jnp.float32),
        pltpu.VMEM((ECHUNK,), jnp.int32),
        pltpu.VMEM((ECHUNK, 16), jnp.float32),
        pltpu.VMEM((RPS, 16), jnp.float32),
        pltpu.VMEM_SHARED((N, 16), jnp.float32),
        pltpu.SemaphoreType.DMA,
    ],
)


def _sc_layer_body(hl_hbm, dinv16_hbm, col_hbm, row_hbm, combo_hbm, T_hbm,
                   agg_out, colv, rowv, combov, gbuf, bbuf, svbuf, mbuf, wb,
                   agg_sh, sem, sem2, sem3):
    c = lax.axis_index("c")
    s = lax.axis_index("s")
    wid = c * NS + s

    def _z(i, _):
        for d in range(8):
            wb[i, pl.ds(d * 16, 16)] = jnp.zeros((16,), jnp.float32)
        return 0
    lax.fori_loop(0, 125, _z, 0)
    for k in range(5):
        pltpu.sync_copy(wb, agg_sh.at[pl.ds(s * RPS + k * 125, 125), :])
    plsc.subcore_barrier()

    def _edges(i, _):
        ebase = wid * EPT + i * ECHUNK
        pltpu.sync_copy(col_hbm.at[pl.ds(ebase, ECHUNK)], colv)
        pltpu.sync_copy(combo_hbm.at[pl.ds(ebase, ECHUNK)], combov)
        pltpu.sync_copy(row_hbm.at[pl.ds(ebase, ECHUNK)], rowv)
        cp1 = pltpu.async_copy(hl_hbm.at[colv], gbuf, sem)
        cp2 = pltpu.async_copy(T_hbm.at[combov], bbuf, sem2)
        cp3 = pltpu.async_copy(dinv16_hbm.at[colv], svbuf, sem3)
        cp1.wait()
        cp2.wait()
        cp3.wait()

        def _edge(j, _):
            sv = svbuf[j, :]
            for d in range(8):
                u = (gbuf[j, pl.ds(d * 16, 16)]
                     + bbuf[j, pl.ds(d * 16, 16)]) * sv
                mbuf[j, pl.ds(d * 16, 16)] = jnp.maximum(u, 0.0)
            return 0
        lax.fori_loop(0, ECHUNK, _edge, 0)
        pltpu.sync_copy(mbuf, agg_sh.at[rowv], add=True)
        return 0
    lax.fori_loop(0, NECHUNK, _edges, 0)
    plsc.subcore_barrier()

    for k in range(5):
        pltpu.sync_copy(agg_sh.at[pl.ds(s * RPS + k * 125, 125), :], wb)
        pltpu.sync_copy(wb, agg_out.at[c, pl.ds(s * RPS + k * 125, 125), :])


_sc_layer = pl.kernel(
    _sc_layer_body,
    out_type=jax.ShapeDtypeStruct((NC, N, D), jnp.float32),
    mesh=_mesh,
    scratch_types=[
        pltpu.VMEM((ECHUNK,), jnp.int32),
        pltpu.VMEM((ECHUNK,), jnp.int32),
        pltpu.VMEM((ECHUNK,), jnp.int32),
        pltpu.VMEM((ECHUNK, D), jnp.float32),
        pltpu.VMEM((ECHUNK, D), jnp.float32),
        pltpu.VMEM((ECHUNK, 16), jnp.float32),
        pltpu.VMEM((ECHUNK, D), jnp.float32),
        pltpu.VMEM((125, D), jnp.float32),
        pltpu.VMEM_SHARED((N, D), jnp.float32),
        pltpu.SemaphoreType.DMA,
        pltpu.SemaphoreType.DMA,
        pltpu.SemaphoreType.DMA,
    ],
)


def _tc_prep_kernel(cnt_ref, h0_ref, w_ref, b_ref, hl_ref, dinv16_ref):
    cnt = cnt_ref[0, :, 0:1] + cnt_ref[1, :, 0:1]
    dinv = lax.rsqrt(cnt + 1.0)
    hl = jnp.dot(h0_ref[...], w_ref[...],
                 preferred_element_type=jnp.float32) + b_ref[...]
    hl_ref[...] = hl
    dinv16_ref[...] = jnp.broadcast_to(dinv, (N, 16))


_tc_prep = pl.pallas_call(
    _tc_prep_kernel,
    out_shape=(jax.ShapeDtypeStruct((N, D), jnp.float32),
               jax.ShapeDtypeStruct((N, 16), jnp.float32)),
)


def _tc_layer_kernel(agg_ref, hl_ref, dinv16_ref, root_ref, g_ref, bt_ref,
                     w_ref, b_ref, out_ref):
    dinv = dinv16_ref[:, 0:1]
    hl = hl_ref[...]
    u = agg_ref[0] + agg_ref[1]
    out = dinv * u + jnp.maximum(hl + root_ref[...], 0.0) * (dinv * dinv)
    mu = jnp.mean(out, axis=0, keepdims=True)
    var = jnp.mean((out - mu) ** 2, axis=0, keepdims=True)
    out = (out - mu) / jnp.sqrt(var + 1e-5) * g_ref[...] + bt_ref[...]
    out = jnp.maximum(out, 0.0)
    out_ref[...] = jnp.dot(out, w_ref[...],
                           preferred_element_type=jnp.float32) + b_ref[...]


_tc_layer = pl.pallas_call(
    _tc_layer_kernel,
    out_shape=jax.ShapeDtypeStruct((N, D), jnp.float32),
)


def _tc_final_kernel(agg_ref, hl_ref, dinv16_ref, root_ref, g_ref, bt_ref,
                     batch_ref, pw_ref, pb_ref, out_ref):
    dinv = dinv16_ref[:, 0:1]
    hl = hl_ref[...]
    u = agg_ref[0] + agg_ref[1]
    out = dinv * u + jnp.maximum(hl + root_ref[...], 0.0) * (dinv * dinv)
    mu = jnp.mean(out, axis=0, keepdims=True)
    var = jnp.mean((out - mu) ** 2, axis=0, keepdims=True)
    out = (out - mu) / jnp.sqrt(var + 1e-5) * g_ref[...] + bt_ref[...]
    gid = lax.broadcasted_iota(jnp.int32, (N, NGRAPHS), 1)
    oh = jnp.where(batch_ref[...] == gid, 1.0, 0.0).astype(jnp.float32)
    hg = lax.dot_general(oh, out, (((0,), (0,)), ((), ())),
                         preferred_element_type=jnp.float32)
    out_ref[...] = jnp.dot(hg, pw_ref[...],
                           preferred_element_type=jnp.float32) + pb_ref[...]


_tc_final = pl.pallas_call(
    _tc_final_kernel,
    out_shape=jax.ShapeDtypeStruct((NGRAPHS, NGRAPHS), jnp.float32),
)


def kernel(x, edge_index, edge_attr, batch, atom_tab, lin_W, lin_b, root,
           bond_tab, bn_gamma, bn_beta, pred_W, pred_b):
    i32 = jnp.int32
    row = edge_index[0].astype(i32)
    col = edge_index[1].astype(i32)
    combo = (edge_attr[:, 0] * 64 + edge_attr[:, 1] * 8
             + edge_attr[:, 2]).astype(i32)
    aidx = (x.astype(i32) + 64 * jnp.arange(9, dtype=i32)[None, :]).reshape(-1)
    atom_flat = atom_tab.reshape(9 * 64, D)
    T = (bond_tab[:, 0][:, :, None, None, :]
         + bond_tab[:, 1][:, None, :, None, :]
         + bond_tab[:, 2][:, None, None, :, :]).reshape(NLAYERS, 512, D)

    h0, cnt = _sc_encode(aidx, row, atom_flat)
    hl, dinv16 = _tc_prep(cnt, h0, lin_W[0], lin_b[0].reshape(1, D))
    for l in range(NLAYERS):
        aggP = _sc_layer(hl, dinv16, col, row, combo, T[l])
        if l < NLAYERS - 1:
            hl = _tc_layer(aggP, hl, dinv16, root[l].reshape(1, D),
                           bn_gamma[l].reshape(1, D), bn_beta[l].reshape(1, D),
                           lin_W[l + 1], lin_b[l + 1].reshape(1, D))
        else:
            out = _tc_final(aggP, hl, dinv16, root[l].reshape(1, D),
                            bn_gamma[l].reshape(1, D), bn_beta[l].reshape(1, D),
                            batch.astype(i32).reshape(N, 1), pred_W,
                            pred_b.reshape(1, NGRAPHS))
    return out


# speedup vs baseline: 6.8213x; 6.8213x over previous
import jax
import jax.numpy as jnp
from jax import lax
from jax.experimental import pallas as pl
from jax.experimental.pallas import tpu as pltpu
from jax.experimental.pallas import tpu_sc as plsc

N = 10000
E = 320000
D = 128
NLAYERS = 5
NGRAPHS = 128

NC = 2
NS = 16
NW = NC * NS

EPT = E // NW
ECHUNK = 80
NECHUNK = EPT // ECHUNK
NPT = 320
NP = 10240
RPS = NP // NS

_mesh = plsc.VectorSubcoreMesh(core_axis_name="c", subcore_axis_name="s",
                               num_cores=NC, num_subcores=NS)


def _sc_encode_body(aidx_hbm, row_hbm, atom_hbm, h0_out, cnt_out,
                    idx_v, gbuf, hbuf, rowv, ones_v, zbuf, cnt_sh, sem):
    c = lax.axis_index("c")
    s = lax.axis_index("s")
    wid = c * NS + s

    def _z(i, _):
        for d in range(8):
            zbuf[i, pl.ds(d * 16, 16)] = jnp.zeros((16,), jnp.float32)
        return 0
    lax.fori_loop(0, 80, _z, 0)
    for k in range(8):
        pltpu.sync_copy(zbuf, cnt_sh.at[pl.ds(s * RPS + k * 80, 80), :])

    def _o(i, _):
        for d in range(8):
            ones_v[i, pl.ds(d * 16, 16)] = jnp.ones((16,), jnp.float32)
        return 0
    lax.fori_loop(0, ECHUNK, _o, 0)
    plsc.subcore_barrier()

    nnodes = jnp.maximum(0, jnp.minimum(NPT, N - wid * NPT))
    nch = nnodes // 8

    def _atom(i, _):
        nbase = wid * NPT + i * 8
        pltpu.sync_copy(aidx_hbm.at[pl.ds(nbase * 9, 72)], idx_v)
        pltpu.async_copy(atom_hbm.at[idx_v], gbuf, sem).wait()
        for j in range(8):
            for d in range(8):
                acc = gbuf[j * 9, pl.ds(d * 16, 16)]
                for k in range(1, 9):
                    acc = acc + gbuf[j * 9 + k, pl.ds(d * 16, 16)]
                hbuf[j, pl.ds(d * 16, 16)] = acc
        pltpu.sync_copy(hbuf, h0_out.at[pl.ds(nbase, 8), :])
        return 0
    lax.fori_loop(0, nch, _atom, 0)

    def _deg(i, _):
        ebase = wid * EPT + i * ECHUNK
        pltpu.sync_copy(row_hbm.at[pl.ds(ebase, ECHUNK)], rowv)
        pltpu.sync_copy(ones_v, cnt_sh.at[rowv], add=True)
        return 0
    lax.fori_loop(0, NECHUNK, _deg, 0)
    plsc.subcore_barrier()

    for k in range(8):
        pltpu.sync_copy(cnt_sh.at[pl.ds(s * RPS + k * 80, 80), :], zbuf)
        pltpu.sync_copy(zbuf, cnt_out.at[c, pl.ds(s * RPS + k * 80, 80), :])


_sc_encode_scratch = [
    pltpu.VMEM((72,), jnp.int32),
    pltpu.VMEM((72, D), jnp.float32),
    pltpu.VMEM((8, D), jnp.float32),
    pltpu.VMEM((ECHUNK,), jnp.int32),
    pltpu.VMEM((ECHUNK, D), jnp.float32),
    pltpu.VMEM((80, D), jnp.float32),
    pltpu.VMEM_SHARED((NP, D), jnp.float32),
    pltpu.SemaphoreType.DMA,
]

_sc_encode = pl.kernel(
    _sc_encode_body,
    out_type=(jax.ShapeDtypeStruct((N, D), jnp.float32),
              jax.ShapeDtypeStruct((NC, NP, D), jnp.float32)),
    mesh=_mesh,
    compiler_params=pltpu.CompilerParams(needs_layout_passes=False),
    scratch_types=_sc_encode_scratch,
)


def _sc_layer_body(hl_hbm, dinv80_hbm, col_hbm, row_hbm, combo_hbm, T_hbm,
                   agg_out, colv, rowv, combov, gbuf, bbuf, mbuf,
                   dinv_v, tmp16, agg_sh, sem, sem2):
    c = lax.axis_index("c")
    s = lax.axis_index("s")
    wid = c * NS + s

    pltpu.sync_copy(dinv80_hbm, dinv_v)

    def _z(i, _):
        for d in range(8):
            mbuf[i, pl.ds(d * 16, 16)] = jnp.zeros((16,), jnp.float32)
        return 0
    lax.fori_loop(0, 80, _z, 0)
    for k in range(8):
        pltpu.sync_copy(mbuf, agg_sh.at[pl.ds(s * RPS + k * 80, 80), :])
    plsc.subcore_barrier()

    def _edges(i, _):
        ebase = wid * EPT + i * ECHUNK
        pltpu.sync_copy(col_hbm.at[pl.ds(ebase, ECHUNK)], colv)
        pltpu.sync_copy(combo_hbm.at[pl.ds(ebase, ECHUNK)], combov)
        pltpu.sync_copy(row_hbm.at[pl.ds(ebase, ECHUNK)], rowv)
        cp1 = pltpu.async_copy(hl_hbm.at[colv], gbuf, sem)
        cp2 = pltpu.async_copy(T_hbm.at[combov], bbuf, sem2)
        cp1.wait()
        cp2.wait()

        def _grp(g, _):
            cv = colv[pl.ds(g * 16, 16)]
            dv = plsc.load_gather(
                dinv_v, [lax.shift_right_logical(cv, 7),
                         jnp.bitwise_and(cv, 127)])
            tmp16[pl.ds(16, 16)] = dv
            for j in range(16):
                sv = plsc.load_gather(tmp16,
                                      [jnp.full((16,), 16 + j, jnp.int32)])
                jj = g * 16 + j
                for d in range(8):
                    u = (gbuf[jj, pl.ds(d * 16, 16)]
                         + bbuf[jj, pl.ds(d * 16, 16)]) * sv
                    mbuf[jj, pl.ds(d * 16, 16)] = jnp.maximum(u, 0.0)
            return 0
        lax.fori_loop(0, ECHUNK // 16, _grp, 0)
        pltpu.sync_copy(mbuf, agg_sh.at[rowv], add=True)
        return 0
    lax.fori_loop(0, NECHUNK, _edges, 0)
    plsc.subcore_barrier()

    for k in range(8):
        pltpu.sync_copy(agg_sh.at[pl.ds(s * RPS + k * 80, 80), :], mbuf)
        pltpu.sync_copy(mbuf, agg_out.at[c, pl.ds(s * RPS + k * 80, 80), :])


_sc_layer_scratch = [
    pltpu.VMEM((ECHUNK,), jnp.int32),
    pltpu.VMEM((ECHUNK,), jnp.int32),
    pltpu.VMEM((ECHUNK,), jnp.int32),
    pltpu.VMEM((ECHUNK, D), jnp.float32),
    pltpu.VMEM((ECHUNK, D), jnp.float32),
    pltpu.VMEM((ECHUNK, D), jnp.float32),
    pltpu.VMEM((80, D), jnp.float32),
    pltpu.VMEM((32,), jnp.float32),
    pltpu.VMEM_SHARED((NP, D), jnp.float32),
    pltpu.SemaphoreType.DMA,
    pltpu.SemaphoreType.DMA,
]

_sc_layer = pl.kernel(
    _sc_layer_body,
    out_type=jax.ShapeDtypeStruct((NC, NP, D), jnp.float32),
    mesh=_mesh,
    compiler_params=pltpu.CompilerParams(needs_layout_passes=False),
    scratch_types=_sc_layer_scratch,
)


def _tc_prep_kernel(cnt_ref, h0_ref, w_ref, b_ref, hl_ref, dinv_ref):
    cnt = cnt_ref[0, :, 0:1] + cnt_ref[1, :, 0:1]
    dinv_ref[...] = lax.rsqrt(cnt + 1.0)
    hl_ref[...] = jnp.dot(h0_ref[...], w_ref[...],
                          preferred_element_type=jnp.float32) + b_ref[...]


_tc_prep = pl.pallas_call(
    _tc_prep_kernel,
    out_shape=(jax.ShapeDtypeStruct((N, D), jnp.float32),
               jax.ShapeDtypeStruct((NP, 1), jnp.float32)),
)


def _tc_layer_kernel(agg_ref, hl_ref, dinv_ref, root_ref, g_ref, bt_ref,
                     w_ref, b_ref, out_ref):
    dinv = dinv_ref[:N, :]
    hl = hl_ref[...]
    u = agg_ref[0, :N, :] + agg_ref[1, :N, :]
    out = dinv * u + jnp.maximum(hl + root_ref[...], 0.0) * (dinv * dinv)
    mu = jnp.mean(out, axis=0, keepdims=True)
    var = jnp.mean((out - mu) ** 2, axis=0, keepdims=True)
    out = (out - mu) / jnp.sqrt(var + 1e-5) * g_ref[...] + bt_ref[...]
    out = jnp.maximum(out, 0.0)
    out_ref[...] = jnp.dot(out, w_ref[...],
                           preferred_element_type=jnp.float32) + b_ref[...]


_tc_layer = pl.pallas_call(
    _tc_layer_kernel,
    out_shape=jax.ShapeDtypeStruct((N, D), jnp.float32),
)


def _tc_final_kernel(agg_ref, hl_ref, dinv_ref, root_ref, g_ref, bt_ref,
                     batch_ref, pw_ref, pb_ref, out_ref):
    dinv = dinv_ref[:N, :]
    hl = hl_ref[...]
    u = agg_ref[0, :N, :] + agg_ref[1, :N, :]
    out = dinv * u + jnp.maximum(hl + root_ref[...], 0.0) * (dinv * dinv)
    mu = jnp.mean(out, axis=0, keepdims=True)
    var = jnp.mean((out - mu) ** 2, axis=0, keepdims=True)
    out = (out - mu) / jnp.sqrt(var + 1e-5) * g_ref[...] + bt_ref[...]
    gid = lax.broadcasted_iota(jnp.int32, (N, NGRAPHS), 1)
    oh = jnp.where(batch_ref[...] == gid, 1.0, 0.0).astype(jnp.float32)
    hg = lax.dot_general(oh, out, (((0,), (0,)), ((), ())),
                         preferred_element_type=jnp.float32)
    out_ref[...] = jnp.dot(hg, pw_ref[...],
                           preferred_element_type=jnp.float32) + pb_ref[...]


_tc_final = pl.pallas_call(
    _tc_final_kernel,
    out_shape=jax.ShapeDtypeStruct((NGRAPHS, NGRAPHS), jnp.float32),
)


def kernel(x, edge_index, edge_attr, batch, atom_tab, lin_W, lin_b, root,
           bond_tab, bn_gamma, bn_beta, pred_W, pred_b):
    i32 = jnp.int32
    row = edge_index[0].astype(i32)
    col = edge_index[1].astype(i32)
    combo = (edge_attr[:, 0] * 64 + edge_attr[:, 1] * 8
             + edge_attr[:, 2]).astype(i32)
    aidx = (x.astype(i32) + 64 * jnp.arange(9, dtype=i32)[None, :]).reshape(-1)
    atom_flat = atom_tab.reshape(9 * 64, D)
    T = (bond_tab[:, 0][:, :, None, None, :]
         + bond_tab[:, 1][:, None, :, None, :]
         + bond_tab[:, 2][:, None, None, :, :]).reshape(NLAYERS, 512, D)

    h0, cnt = _sc_encode(aidx, row, atom_flat)
    hl, dinvc = _tc_prep(cnt, h0, lin_W[0], lin_b[0].reshape(1, D))
    dinv80 = dinvc.reshape(NP // D, D)
    for l in range(NLAYERS):
        aggP = _sc_layer(hl, dinv80, col, row, combo, T[l])
        if l < NLAYERS - 1:
            hl = _tc_layer(aggP, hl, dinvc, root[l].reshape(1, D),
                           bn_gamma[l].reshape(1, D), bn_beta[l].reshape(1, D),
                           lin_W[l + 1], lin_b[l + 1].reshape(1, D))
        else:
            out = _tc_final(aggP, hl, dinvc, root[l].reshape(1, D),
                            bn_gamma[l].reshape(1, D), bn_beta[l].reshape(1, D),
                            batch.astype(i32).reshape(N, 1), pred_W,
                            pred_b.reshape(1, NGRAPHS))
    return out


# speedup vs baseline: 7.5383x; 1.1051x over previous
import jax
import jax.numpy as jnp
from jax import lax
from jax.experimental import pallas as pl
from jax.experimental.pallas import tpu as pltpu
from jax.experimental.pallas import tpu_sc as plsc

N = 10000
E = 320000
D = 128
NLAYERS = 5
NGRAPHS = 128

NC = 2
NS = 16
NW = NC * NS

EPT = E // NW
ECHUNK = 40
NECHUNK = EPT // ECHUNK
NPAIR = NECHUNK // 2
ECH_ENC = 80
NECH_ENC = EPT // ECH_ENC
NPT = 320
NP = 10240
RPS = NP // NS

_mesh = plsc.VectorSubcoreMesh(core_axis_name="c", subcore_axis_name="s",
                               num_cores=NC, num_subcores=NS)


def _sc_encode_body(aidx_hbm, row_hbm, atom_hbm, h0_out, cnt_out,
                    idx_v, gbuf, hbuf, rowv, ones_v, zbuf, cnt_sh, sem):
    c = lax.axis_index("c")
    s = lax.axis_index("s")
    wid = c * NS + s

    def _z(i, _):
        for d in range(8):
            zbuf[i, pl.ds(d * 16, 16)] = jnp.zeros((16,), jnp.float32)
        return 0
    lax.fori_loop(0, 80, _z, 0)
    for k in range(8):
        pltpu.sync_copy(zbuf, cnt_sh.at[pl.ds(s * RPS + k * 80, 80), :])

    def _o(i, _):
        for d in range(8):
            ones_v[i, pl.ds(d * 16, 16)] = jnp.ones((16,), jnp.float32)
        return 0
    lax.fori_loop(0, ECH_ENC, _o, 0)
    plsc.subcore_barrier()

    nnodes = jnp.maximum(0, jnp.minimum(NPT, N - wid * NPT))
    nch = nnodes // 8

    def _atom(i, _):
        nbase = wid * NPT + i * 8
        pltpu.sync_copy(aidx_hbm.at[pl.ds(nbase * 9, 72)], idx_v)
        pltpu.async_copy(atom_hbm.at[idx_v], gbuf, sem).wait()
        for j in range(8):
            for d in range(8):
                acc = gbuf[j * 9, pl.ds(d * 16, 16)]
                for k in range(1, 9):
                    acc = acc + gbuf[j * 9 + k, pl.ds(d * 16, 16)]
                hbuf[j, pl.ds(d * 16, 16)] = acc
        pltpu.sync_copy(hbuf, h0_out.at[pl.ds(nbase, 8), :])
        return 0
    lax.fori_loop(0, nch, _atom, 0)

    def _deg(i, _):
        ebase = wid * EPT + i * ECH_ENC
        pltpu.sync_copy(row_hbm.at[pl.ds(ebase, ECH_ENC)], rowv)
        pltpu.sync_copy(ones_v, cnt_sh.at[rowv], add=True)
        return 0
    lax.fori_loop(0, NECH_ENC, _deg, 0)
    plsc.subcore_barrier()

    for k in range(8):
        pltpu.sync_copy(cnt_sh.at[pl.ds(s * RPS + k * 80, 80), :], zbuf)
        pltpu.sync_copy(zbuf, cnt_out.at[c, pl.ds(s * RPS + k * 80, 80), :])


_sc_encode_scratch = [
    pltpu.VMEM((72,), jnp.int32),
    pltpu.VMEM((72, D), jnp.float32),
    pltpu.VMEM((8, D), jnp.float32),
    pltpu.VMEM((ECH_ENC,), jnp.int32),
    pltpu.VMEM((ECH_ENC, D), jnp.float32),
    pltpu.VMEM((80, D), jnp.float32),
    pltpu.VMEM_SHARED((NP, D), jnp.float32),
    pltpu.SemaphoreType.DMA,
]

_sc_encode = pl.kernel(
    _sc_encode_body,
    out_type=(jax.ShapeDtypeStruct((N, D), jnp.float32),
              jax.ShapeDtypeStruct((NC, NP, D), jnp.float32)),
    mesh=_mesh,
    compiler_params=pltpu.CompilerParams(needs_layout_passes=False),
    scratch_types=_sc_encode_scratch,
)


def _sc_layer_body(hl_hbm, dinv80_hbm, col_hbm, row_hbm, combo_hbm, T_hbm,
                   agg_out, colvA, rowvA, combovA, colvB, rowvB, combovB,
                   gA, bA, mA, gB, bB, mB, dinv_v, tmp32, agg_sh,
                   sgA1, sgA2, ssA, sgB1, sgB2, ssB):
    c = lax.axis_index("c")
    s = lax.axis_index("s")
    wid = c * NS + s

    pltpu.sync_copy(dinv80_hbm, dinv_v)

    def _z(i, _):
        for d in range(8):
            mA[i, pl.ds(d * 16, 16)] = jnp.zeros((16,), jnp.float32)
        return 0
    lax.fori_loop(0, ECHUNK, _z, 0)
    for k in range(RPS // ECHUNK):
        pltpu.sync_copy(mA, agg_sh.at[pl.ds(s * RPS + k * ECHUNK, ECHUNK), :])
    plsc.subcore_barrier()

    def _eb(cc):
        return wid * EPT + cc * ECHUNK

    def _load_cc(cc, colv, combov):
        pltpu.sync_copy(col_hbm.at[pl.ds(_eb(cc), ECHUNK)], colv)
        pltpu.sync_copy(combo_hbm.at[pl.ds(_eb(cc), ECHUNK)], combov)

    def _load_row(cc, rowv):
        pltpu.sync_copy(row_hbm.at[pl.ds(_eb(cc), ECHUNK)], rowv)

    def _start_g(colv, combov, g, b, s1, s2):
        pltpu.async_copy(hl_hbm.at[colv], g, s1)
        pltpu.async_copy(T_hbm.at[combov], b, s2)

    def _wait_g(colv, combov, g, b, s1, s2):
        pltpu.make_async_copy(hl_hbm.at[colv], g, s1).wait()
        pltpu.make_async_copy(T_hbm.at[combov], b, s2).wait()

    def _compute(colv, g, b, m):
        for base, jlo in ((0, 0), (16, 0), (24, 8)):
            cv = colv[pl.ds(base, 16)]
            dv = plsc.load_gather(
                dinv_v, [lax.shift_right_logical(cv, 7),
                         jnp.bitwise_and(cv, 127)])
            tmp32[pl.ds(16, 16)] = dv
            for j in range(jlo, 16):
                sv = plsc.load_gather(tmp32,
                                      [jnp.full((16,), 16 + j, jnp.int32)])
                jj = base + j
                for d in range(8):
                    u = (g[jj, pl.ds(d * 16, 16)]
                         + b[jj, pl.ds(d * 16, 16)]) * sv
                    m[jj, pl.ds(d * 16, 16)] = jnp.maximum(u, 0.0)

    _load_cc(0, colvA, combovA)
    _start_g(colvA, combovA, gA, bA, sgA1, sgA2)
    _load_row(0, rowvA)
    _load_cc(1, colvB, combovB)
    _start_g(colvB, combovB, gB, bB, sgB1, sgB2)
    _load_row(1, rowvB)

    def _iter(i, _):
        cA = 2 * i
        cB = 2 * i + 1

        @pl.when(i > 0)
        def _():
            pltpu.make_async_copy(mA, agg_sh.at[rowvA], ssA).wait()
            _load_row(cA, rowvA)
        _wait_g(colvA, combovA, gA, bA, sgA1, sgA2)
        _compute(colvA, gA, bA, mA)
        pltpu.async_copy(mA, agg_sh.at[rowvA], ssA, add=True)

        @pl.when(i < NPAIR - 1)
        def _():
            _load_cc(cA + 2, colvA, combovA)
            _start_g(colvA, combovA, gA, bA, sgA1, sgA2)

        @pl.when(i > 0)
        def _():
            pltpu.make_async_copy(mB, agg_sh.at[rowvB], ssB).wait()
            _load_row(cB, rowvB)
        _wait_g(colvB, combovB, gB, bB, sgB1, sgB2)
        _compute(colvB, gB, bB, mB)
        pltpu.async_copy(mB, agg_sh.at[rowvB], ssB, add=True)

        @pl.when(i < NPAIR - 1)
        def _():
            _load_cc(cB + 2, colvB, combovB)
            _start_g(colvB, combovB, gB, bB, sgB1, sgB2)
        return 0
    lax.fori_loop(0, NPAIR, _iter, 0)
    pltpu.make_async_copy(mA, agg_sh.at[rowvA], ssA).wait()
    pltpu.make_async_copy(mB, agg_sh.at[rowvB], ssB).wait()
    plsc.subcore_barrier()

    pltpu.sync_copy(agg_sh.at[pl.ds(s * RPS, RPS), :],
                    agg_out.at[c, pl.ds(s * RPS, RPS), :])


_sc_layer_scratch = [
    pltpu.VMEM((ECHUNK,), jnp.int32),
    pltpu.VMEM((ECHUNK,), jnp.int32),
    pltpu.VMEM((ECHUNK,), jnp.int32),
    pltpu.VMEM((ECHUNK,), jnp.int32),
    pltpu.VMEM((ECHUNK,), jnp.int32),
    pltpu.VMEM((ECHUNK,), jnp.int32),
    pltpu.VMEM((ECHUNK, D), jnp.float32),
    pltpu.VMEM((ECHUNK, D), jnp.float32),
    pltpu.VMEM((ECHUNK, D), jnp.float32),
    pltpu.VMEM((ECHUNK, D), jnp.float32),
    pltpu.VMEM((ECHUNK, D), jnp.float32),
    pltpu.VMEM((ECHUNK, D), jnp.float32),
    pltpu.VMEM((80, D), jnp.float32),
    pltpu.VMEM((32,), jnp.float32),
    pltpu.VMEM_SHARED((NP, D), jnp.float32),
    pltpu.SemaphoreType.DMA,
    pltpu.SemaphoreType.DMA,
    pltpu.SemaphoreType.DMA,
    pltpu.SemaphoreType.DMA,
    pltpu.SemaphoreType.DMA,
    pltpu.SemaphoreType.DMA,
]

_sc_layer = pl.kernel(
    _sc_layer_body,
    out_type=jax.ShapeDtypeStruct((NC, NP, D), jnp.float32),
    mesh=_mesh,
    compiler_params=pltpu.CompilerParams(needs_layout_passes=False),
    scratch_types=_sc_layer_scratch,
)


def _tc_prep_kernel(cnt_ref, h0_ref, w_ref, b_ref, hl_ref, dinv_ref):
    cnt = cnt_ref[0, :, 0:1] + cnt_ref[1, :, 0:1]
    dinv_ref[...] = lax.rsqrt(cnt + 1.0)
    hl_ref[...] = jnp.dot(h0_ref[...], w_ref[...],
                          preferred_element_type=jnp.float32) + b_ref[...]


_tc_prep = pl.pallas_call(
    _tc_prep_kernel,
    out_shape=(jax.ShapeDtypeStruct((N, D), jnp.float32),
               jax.ShapeDtypeStruct((NP, 1), jnp.float32)),
)


def _tc_layer_kernel(agg_ref, hl_ref, dinv_ref, root_ref, g_ref, bt_ref,
                     w_ref, b_ref, out_ref):
    dinv = dinv_ref[:N, :]
    hl = hl_ref[...]
    u = agg_ref[0, :N, :] + agg_ref[1, :N, :]
    out = dinv * u + jnp.maximum(hl + root_ref[...], 0.0) * (dinv * dinv)
    mu = jnp.mean(out, axis=0, keepdims=True)
    var = jnp.mean((out - mu) ** 2, axis=0, keepdims=True)
    out = (out - mu) / jnp.sqrt(var + 1e-5) * g_ref[...] + bt_ref[...]
    out = jnp.maximum(out, 0.0)
    out_ref[...] = jnp.dot(out, w_ref[...],
                           preferred_element_type=jnp.float32) + b_ref[...]


_tc_layer = pl.pallas_call(
    _tc_layer_kernel,
    out_shape=jax.ShapeDtypeStruct((N, D), jnp.float32),
)


def _tc_final_kernel(agg_ref, hl_ref, dinv_ref, root_ref, g_ref, bt_ref,
                     batch_ref, pw_ref, pb_ref, out_ref):
    dinv = dinv_ref[:N, :]
    hl = hl_ref[...]
    u = agg_ref[0, :N, :] + agg_ref[1, :N, :]
    out = dinv * u + jnp.maximum(hl + root_ref[...], 0.0) * (dinv * dinv)
    mu = jnp.mean(out, axis=0, keepdims=True)
    var = jnp.mean((out - mu) ** 2, axis=0, keepdims=True)
    out = (out - mu) / jnp.sqrt(var + 1e-5) * g_ref[...] + bt_ref[...]
    gid = lax.broadcasted_iota(jnp.int32, (N, NGRAPHS), 1)
    oh = jnp.where(batch_ref[...] == gid, 1.0, 0.0).astype(jnp.float32)
    hg = lax.dot_general(oh, out, (((0,), (0,)), ((), ())),
                         preferred_element_type=jnp.float32)
    out_ref[...] = jnp.dot(hg, pw_ref[...],
                           preferred_element_type=jnp.float32) + pb_ref[...]


_tc_final = pl.pallas_call(
    _tc_final_kernel,
    out_shape=jax.ShapeDtypeStruct((NGRAPHS, NGRAPHS), jnp.float32),
)


def kernel(x, edge_index, edge_attr, batch, atom_tab, lin_W, lin_b, root,
           bond_tab, bn_gamma, bn_beta, pred_W, pred_b):
    i32 = jnp.int32
    row = edge_index[0].astype(i32)
    col = edge_index[1].astype(i32)
    combo = (edge_attr[:, 0] * 64 + edge_attr[:, 1] * 8
             + edge_attr[:, 2]).astype(i32)
    aidx = (x.astype(i32) + 64 * jnp.arange(9, dtype=i32)[None, :]).reshape(-1)
    atom_flat = atom_tab.reshape(9 * 64, D)
    T = (bond_tab[:, 0][:, :, None, None, :]
         + bond_tab[:, 1][:, None, :, None, :]
         + bond_tab[:, 2][:, None, None, :, :]).reshape(NLAYERS, 512, D)

    h0, cnt = _sc_encode(aidx, row, atom_flat)
    hl, dinvc = _tc_prep(cnt, h0, lin_W[0], lin_b[0].reshape(1, D))
    dinv80 = dinvc.reshape(NP // D, D)
    for l in range(NLAYERS):
        aggP = _sc_layer(hl, dinv80, col, row, combo, T[l])
        if l < NLAYERS - 1:
            hl = _tc_layer(aggP, hl, dinvc, root[l].reshape(1, D),
                           bn_gamma[l].reshape(1, D), bn_beta[l].reshape(1, D),
                           lin_W[l + 1], lin_b[l + 1].reshape(1, D))
        else:
            out = _tc_final(aggP, hl, dinvc, root[l].reshape(1, D),
                            bn_gamma[l].reshape(1, D), bn_beta[l].reshape(1, D),
                            batch.astype(i32).reshape(N, 1), pred_W,
                            pred_b.reshape(1, NGRAPHS))
    return out


# speedup vs baseline: 11.5076x; 1.5266x over previous
import jax
import jax.numpy as jnp
from jax import lax
from jax.experimental import pallas as pl
from jax.experimental.pallas import tpu as pltpu
from jax.experimental.pallas import tpu_sc as plsc

N = 10000
E = 320000
D = 128
NLAYERS = 5
NGRAPHS = 128

NC = 2
NS = 16
NW = NC * NS

EPT = E // NW
ECHUNK = 40
NECHUNK = EPT // ECHUNK
NPAIR = NECHUNK // 2
ECH_ENC = 80
NECH_ENC = EPT // ECH_ENC
NPT = 320
NP = 10240
RPS = NP // NS

_mesh = plsc.VectorSubcoreMesh(core_axis_name="c", subcore_axis_name="s",
                               num_cores=NC, num_subcores=NS)


def _sc_encode_body(aidx_hbm, row_hbm, atom_hbm, h0_out, cnt_out,
                    idx_v, gbuf, hbuf, rowv, ones_v, zbuf, cnt_sh, sem):
    c = lax.axis_index("c")
    s = lax.axis_index("s")
    wid = c * NS + s

    def _z(i, _):
        for d in range(8):
            zbuf[i, pl.ds(d * 16, 16)] = jnp.zeros((16,), jnp.float32)
        return 0
    lax.fori_loop(0, 80, _z, 0)
    for k in range(8):
        pltpu.sync_copy(zbuf, cnt_sh.at[pl.ds(s * RPS + k * 80, 80), :])

    def _o(i, _):
        for d in range(8):
            ones_v[i, pl.ds(d * 16, 16)] = jnp.ones((16,), jnp.float32)
        return 0
    lax.fori_loop(0, ECH_ENC, _o, 0)
    plsc.subcore_barrier()

    nnodes = jnp.maximum(0, jnp.minimum(NPT, N - wid * NPT))
    nch = nnodes // 8

    def _atom(i, _):
        nbase = wid * NPT + i * 8
        pltpu.sync_copy(aidx_hbm.at[pl.ds(nbase * 9, 72)], idx_v)
        pltpu.async_copy(atom_hbm.at[idx_v], gbuf, sem).wait()
        for j in range(8):
            for d in range(8):
                acc = gbuf[j * 9, pl.ds(d * 16, 16)]
                for k in range(1, 9):
                    acc = acc + gbuf[j * 9 + k, pl.ds(d * 16, 16)]
                hbuf[j, pl.ds(d * 16, 16)] = acc
        pltpu.sync_copy(hbuf, h0_out.at[pl.ds(nbase, 8), :])
        return 0
    lax.fori_loop(0, nch, _atom, 0)

    def _deg(i, _):
        ebase = wid * EPT + i * ECH_ENC
        pltpu.sync_copy(row_hbm.at[pl.ds(ebase, ECH_ENC)], rowv)
        pltpu.sync_copy(ones_v, cnt_sh.at[rowv], add=True)
        return 0
    lax.fori_loop(0, NECH_ENC, _deg, 0)
    plsc.subcore_barrier()

    for k in range(8):
        pltpu.sync_copy(cnt_sh.at[pl.ds(s * RPS + k * 80, 80), :], zbuf)
        pltpu.sync_copy(zbuf, cnt_out.at[c, pl.ds(s * RPS + k * 80, 80), :])


_sc_encode_scratch = [
    pltpu.VMEM((72,), jnp.int32),
    pltpu.VMEM((72, D), jnp.float32),
    pltpu.VMEM((8, D), jnp.float32),
    pltpu.VMEM((ECH_ENC,), jnp.int32),
    pltpu.VMEM((ECH_ENC, D), jnp.float32),
    pltpu.VMEM((80, D), jnp.float32),
    pltpu.VMEM_SHARED((NP, D), jnp.float32),
    pltpu.SemaphoreType.DMA,
]

_sc_encode = pl.kernel(
    _sc_encode_body,
    out_type=(jax.ShapeDtypeStruct((N, D), jnp.float32),
              jax.ShapeDtypeStruct((NC, NP, D), jnp.float32)),
    mesh=_mesh,
    compiler_params=pltpu.CompilerParams(needs_layout_passes=False),
    scratch_types=_sc_encode_scratch,
)


def _sc_layer_body(hl_hbm, dinv80_hbm, ccr_hbm, T_hbm,
                   agg_out, ebufA, ebufB, colvA, rowvA, combovA,
                   colvB, rowvB, combovB,
                   gA, bA, mA, gB, bB, mB, dinv_v, tmp32, agg_sh,
                   sgA1, sgA2, ssA, sgB1, sgB2, ssB):
    c = lax.axis_index("c")
    s = lax.axis_index("s")
    wid = c * NS + s

    pltpu.sync_copy(dinv80_hbm, dinv_v)

    def _z(i, _):
        for d in range(8):
            mA[i, pl.ds(d * 16, 16)] = jnp.zeros((16,), jnp.float32)
        return 0
    lax.fori_loop(0, ECHUNK, _z, 0)
    for k in range(RPS // ECHUNK):
        pltpu.sync_copy(mA, agg_sh.at[pl.ds(s * RPS + k * ECHUNK, ECHUNK), :])
    plsc.subcore_barrier()

    def _load_e(cc, ebuf):
        base = (wid * NECHUNK + cc) * 3 * ECHUNK
        pltpu.sync_copy(ccr_hbm.at[pl.ds(base, 3 * ECHUNK)], ebuf)

    def _unpack_cc(ebuf, colv, combov):
        colv[pl.ds(0, 16)] = ebuf[pl.ds(0, 16)]
        colv[pl.ds(16, 16)] = ebuf[pl.ds(16, 16)]
        colv[pl.ds(24, 16)] = ebuf[pl.ds(24, 16)]
        combov[pl.ds(0, 16)] = ebuf[pl.ds(40, 16)]
        combov[pl.ds(16, 16)] = ebuf[pl.ds(56, 16)]
        combov[pl.ds(24, 16)] = ebuf[pl.ds(64, 16)]

    def _unpack_row(ebuf, rowv):
        rowv[pl.ds(0, 16)] = ebuf[pl.ds(80, 16)]
        rowv[pl.ds(16, 16)] = ebuf[pl.ds(96, 16)]
        rowv[pl.ds(24, 16)] = ebuf[pl.ds(104, 16)]

    def _start_g(colv, combov, g, b, s1, s2):
        pltpu.async_copy(hl_hbm.at[colv], g, s1)
        pltpu.async_copy(T_hbm.at[combov], b, s2)

    def _wait_g(colv, combov, g, b, s1, s2):
        pltpu.make_async_copy(hl_hbm.at[colv], g, s1).wait()
        pltpu.make_async_copy(T_hbm.at[combov], b, s2).wait()

    def _compute(colv, g, b, m):
        for base, jlo in ((0, 0), (16, 0), (24, 8)):
            cv = colv[pl.ds(base, 16)]
            dv = plsc.load_gather(
                dinv_v, [lax.shift_right_logical(cv, 7),
                         jnp.bitwise_and(cv, 127)])
            tmp32[pl.ds(16, 16)] = dv
            for j in range(jlo, 16):
                sv = plsc.load_gather(tmp32,
                                      [jnp.full((16,), 16 + j, jnp.int32)])
                jj = base + j
                for d in range(8):
                    u = (g[jj, pl.ds(d * 16, 16)]
                         + b[jj, pl.ds(d * 16, 16)]) * sv
                    m[jj, pl.ds(d * 16, 16)] = jnp.maximum(u, 0.0)

    _load_e(0, ebufA)
    _unpack_cc(ebufA, colvA, combovA)
    _unpack_row(ebufA, rowvA)
    _start_g(colvA, combovA, gA, bA, sgA1, sgA2)
    _load_e(1, ebufB)
    _unpack_cc(ebufB, colvB, combovB)
    _unpack_row(ebufB, rowvB)
    _start_g(colvB, combovB, gB, bB, sgB1, sgB2)

    def _iter(i, _):
        cA = 2 * i
        cB = 2 * i + 1

        @pl.when(i > 0)
        def _():
            pltpu.make_async_copy(mA, agg_sh.at[rowvA], ssA).wait()
            _unpack_row(ebufA, rowvA)
        _wait_g(colvA, combovA, gA, bA, sgA1, sgA2)
        _compute(colvA, gA, bA, mA)
        pltpu.async_copy(mA, agg_sh.at[rowvA], ssA, add=True)

        @pl.when(i < NPAIR - 1)
        def _():
            _load_e(cA + 2, ebufA)
            _unpack_cc(ebufA, colvA, combovA)
            _start_g(colvA, combovA, gA, bA, sgA1, sgA2)

        @pl.when(i > 0)
        def _():
            pltpu.make_async_copy(mB, agg_sh.at[rowvB], ssB).wait()
            _unpack_row(ebufB, rowvB)
        _wait_g(colvB, combovB, gB, bB, sgB1, sgB2)
        _compute(colvB, gB, bB, mB)
        pltpu.async_copy(mB, agg_sh.at[rowvB], ssB, add=True)

        @pl.when(i < NPAIR - 1)
        def _():
            _load_e(cB + 2, ebufB)
            _unpack_cc(ebufB, colvB, combovB)
            _start_g(colvB, combovB, gB, bB, sgB1, sgB2)
        return 0
    lax.fori_loop(0, NPAIR, _iter, 0)
    pltpu.make_async_copy(mA, agg_sh.at[rowvA], ssA).wait()
    pltpu.make_async_copy(mB, agg_sh.at[rowvB], ssB).wait()
    plsc.subcore_barrier()

    pltpu.sync_copy(agg_sh.at[pl.ds(s * RPS, RPS), :],
                    agg_out.at[c, pl.ds(s * RPS, RPS), :])


_sc_layer_scratch = [
    pltpu.VMEM((3 * ECHUNK,), jnp.int32),
    pltpu.VMEM((3 * ECHUNK,), jnp.int32),
    pltpu.VMEM((ECHUNK,), jnp.int32),
    pltpu.VMEM((ECHUNK,), jnp.int32),
    pltpu.VMEM((ECHUNK,), jnp.int32),
    pltpu.VMEM((ECHUNK,), jnp.int32),
    pltpu.VMEM((ECHUNK,), jnp.int32),
    pltpu.VMEM((ECHUNK,), jnp.int32),
    pltpu.VMEM((ECHUNK, D), jnp.float32),
    pltpu.VMEM((ECHUNK, D), jnp.float32),
    pltpu.VMEM((ECHUNK, D), jnp.float32),
    pltpu.VMEM((ECHUNK, D), jnp.float32),
    pltpu.VMEM((ECHUNK, D), jnp.float32),
    pltpu.VMEM((ECHUNK, D), jnp.float32),
    pltpu.VMEM((80, D), jnp.float32),
    pltpu.VMEM((32,), jnp.float32),
    pltpu.VMEM_SHARED((NP, D), jnp.float32),
    pltpu.SemaphoreType.DMA,
    pltpu.SemaphoreType.DMA,
    pltpu.SemaphoreType.DMA,
    pltpu.SemaphoreType.DMA,
    pltpu.SemaphoreType.DMA,
    pltpu.SemaphoreType.DMA,
]

_sc_layer = pl.kernel(
    _sc_layer_body,
    out_type=jax.ShapeDtypeStruct((NC, NP, D), jnp.float32),
    mesh=_mesh,
    compiler_params=pltpu.CompilerParams(needs_layout_passes=False),
    scratch_types=_sc_layer_scratch,
)


def _tc_prep_kernel(cnt_ref, h0_ref, w_ref, b_ref, hl_ref, dinv_ref):
    cnt = cnt_ref[0, :, 0:1] + cnt_ref[1, :, 0:1]
    dinv_ref[...] = lax.rsqrt(cnt + 1.0)
    hl_ref[...] = jnp.dot(h0_ref[...], w_ref[...],
                          preferred_element_type=jnp.float32) + b_ref[...]


_tc_prep = pl.pallas_call(
    _tc_prep_kernel,
    out_shape=(jax.ShapeDtypeStruct((N, D), jnp.float32),
               jax.ShapeDtypeStruct((NP, 1), jnp.float32)),
)


def _tc_layer_kernel(agg_ref, hl_ref, dinv_ref, root_ref, g_ref, bt_ref,
                     w_ref, b_ref, out_ref):
    dinv = dinv_ref[:N, :]
    hl = hl_ref[...]
    u = agg_ref[0, :N, :] + agg_ref[1, :N, :]
    out = dinv * u + jnp.maximum(hl + root_ref[...], 0.0) * (dinv * dinv)
    mu = jnp.mean(out, axis=0, keepdims=True)
    var = jnp.mean((out - mu) ** 2, axis=0, keepdims=True)
    out = (out - mu) / jnp.sqrt(var + 1e-5) * g_ref[...] + bt_ref[...]
    out = jnp.maximum(out, 0.0)
    out_ref[...] = jnp.dot(out, w_ref[...],
                           preferred_element_type=jnp.float32) + b_ref[...]


_tc_layer = pl.pallas_call(
    _tc_layer_kernel,
    out_shape=jax.ShapeDtypeStruct((N, D), jnp.float32),
)


def _tc_final_kernel(agg_ref, hl_ref, dinv_ref, root_ref, g_ref, bt_ref,
                     batch_ref, pw_ref, pb_ref, out_ref):
    dinv = dinv_ref[:N, :]
    hl = hl_ref[...]
    u = agg_ref[0, :N, :] + agg_ref[1, :N, :]
    out = dinv * u + jnp.maximum(hl + root_ref[...], 0.0) * (dinv * dinv)
    mu = jnp.mean(out, axis=0, keepdims=True)
    var = jnp.mean((out - mu) ** 2, axis=0, keepdims=True)
    out = (out - mu) / jnp.sqrt(var + 1e-5) * g_ref[...] + bt_ref[...]
    gid = lax.broadcasted_iota(jnp.int32, (N, NGRAPHS), 1)
    oh = jnp.where(batch_ref[...] == gid, 1.0, 0.0).astype(jnp.float32)
    hg = lax.dot_general(oh, out, (((0,), (0,)), ((), ())),
                         preferred_element_type=jnp.float32)
    out_ref[...] = jnp.dot(hg, pw_ref[...],
                           preferred_element_type=jnp.float32) + pb_ref[...]


_tc_final = pl.pallas_call(
    _tc_final_kernel,
    out_shape=jax.ShapeDtypeStruct((NGRAPHS, NGRAPHS), jnp.float32),
)


def kernel(x, edge_index, edge_attr, batch, atom_tab, lin_W, lin_b, root,
           bond_tab, bn_gamma, bn_beta, pred_W, pred_b):
    i32 = jnp.int32
    row = edge_index[0].astype(i32)
    col = edge_index[1].astype(i32)
    combo = (edge_attr[:, 0] * 64 + edge_attr[:, 1] * 8
             + edge_attr[:, 2]).astype(i32)
    aidx = (x.astype(i32) + 64 * jnp.arange(9, dtype=i32)[None, :]).reshape(-1)
    atom_flat = atom_tab.reshape(9 * 64, D)
    T = (bond_tab[:, 0][:, :, None, None, :]
         + bond_tab[:, 1][:, None, :, None, :]
         + bond_tab[:, 2][:, None, None, :, :]).reshape(NLAYERS, 512, D)

    ccr = jnp.stack([col.reshape(-1, ECHUNK), combo.reshape(-1, ECHUNK),
                     row.reshape(-1, ECHUNK)], axis=1).reshape(-1)
    h0, cnt = _sc_encode(aidx, row, atom_flat)
    hl, dinvc = _tc_prep(cnt, h0, lin_W[0], lin_b[0].reshape(1, D))
    dinv80 = dinvc.reshape(NP // D, D)
    for l in range(NLAYERS):
        aggP = _sc_layer(hl, dinv80, ccr, T[l])
        if l < NLAYERS - 1:
            hl = _tc_layer(aggP, hl, dinvc, root[l].reshape(1, D),
                           bn_gamma[l].reshape(1, D), bn_beta[l].reshape(1, D),
                           lin_W[l + 1], lin_b[l + 1].reshape(1, D))
        else:
            out = _tc_final(aggP, hl, dinvc, root[l].reshape(1, D),
                            bn_gamma[l].reshape(1, D), bn_beta[l].reshape(1, D),
                            batch.astype(i32).reshape(N, 1), pred_W,
                            pred_b.reshape(1, NGRAPHS))
    return out


# speedup vs baseline: 11.7121x; 1.0178x over previous
import jax
import jax.numpy as jnp
from jax import lax
from jax.experimental import pallas as pl
from jax.experimental.pallas import tpu as pltpu
from jax.experimental.pallas import tpu_sc as plsc

N = 10000
E = 320000
D = 128
NLAYERS = 5
NGRAPHS = 128

NC = 2
NS = 16
NW = NC * NS

EPT = E // NW
ECHUNK = 40
NECHUNK = EPT // ECHUNK
NPAIR = NECHUNK // 2
ECH_ENC = 40
NECH_ENC = EPT // ECH_ENC
NPT = 320
NP = 10240
RPS = NP // NS

_mesh = plsc.VectorSubcoreMesh(core_axis_name="c", subcore_axis_name="s",
                               num_cores=NC, num_subcores=NS)


def _sc_encode_body(aidx_hbm, row_hbm, atom_hbm, h0_out, cnt_out,
                    idxA, idxB, gbufA, gbufB, hbuf, rowvA, rowvB, ones_v,
                    zbuf, cnt_sh, semA, semB, srA, srB):
    c = lax.axis_index("c")
    s = lax.axis_index("s")
    wid = c * NS + s

    def _z(i, _):
        for d in range(8):
            zbuf[i, pl.ds(d * 16, 16)] = jnp.zeros((16,), jnp.float32)
        return 0
    lax.fori_loop(0, 80, _z, 0)
    for k in range(8):
        pltpu.sync_copy(zbuf, cnt_sh.at[pl.ds(s * RPS + k * 80, 80), :])

    def _o(i, _):
        for d in range(8):
            ones_v[i, pl.ds(d * 16, 16)] = jnp.ones((16,), jnp.float32)
        return 0
    lax.fori_loop(0, ECH_ENC, _o, 0)
    plsc.subcore_barrier()

    nnodes = jnp.maximum(0, jnp.minimum(NPT, N - wid * NPT))
    nch = nnodes // 8

    def _prefetch(i, idxv, gbuf, sem):
        nbase = wid * NPT + i * 8
        pltpu.sync_copy(aidx_hbm.at[pl.ds(nbase * 9, 72)], idxv)
        pltpu.async_copy(atom_hbm.at[idxv], gbuf, sem)

    def _consume(i, idxv, gbuf, sem):
        nbase = wid * NPT + i * 8
        pltpu.make_async_copy(atom_hbm.at[idxv], gbuf, sem).wait()
        for j in range(8):
            for d in range(8):
                acc = gbuf[j * 9, pl.ds(d * 16, 16)]
                for k in range(1, 9):
                    acc = acc + gbuf[j * 9 + k, pl.ds(d * 16, 16)]
                hbuf[j, pl.ds(d * 16, 16)] = acc
        pltpu.sync_copy(hbuf, h0_out.at[pl.ds(nbase, 8), :])

    _prefetch(0, idxA, gbufA, semA)
    _prefetch(1, idxB, gbufB, semB)

    def _pair(i, _):
        _consume(2 * i, idxA, gbufA, semA)

        @pl.when(2 * i + 2 < nch)
        def _():
            _prefetch(2 * i + 2, idxA, gbufA, semA)
        _consume(2 * i + 1, idxB, gbufB, semB)

        @pl.when(2 * i + 3 < nch)
        def _():
            _prefetch(2 * i + 3, idxB, gbufB, semB)
        return 0
    lax.fori_loop(0, nch // 2, _pair, 0)

    def _ldrow(i, rowv, sem):
        ebase = wid * EPT + i * ECH_ENC
        pltpu.async_copy(row_hbm.at[pl.ds(ebase, ECH_ENC)], rowv, sem)

    _ldrow(0, rowvA, srA)
    _ldrow(1, rowvB, srB)

    def _deg(i, _):
        pltpu.make_async_copy(row_hbm.at[pl.ds(0, ECH_ENC)], rowvA, srA).wait()
        pltpu.sync_copy(ones_v, cnt_sh.at[rowvA], add=True)

        @pl.when(i < NECH_ENC // 2 - 1)
        def _():
            _ldrow(2 * i + 2, rowvA, srA)
        pltpu.make_async_copy(row_hbm.at[pl.ds(0, ECH_ENC)], rowvB, srB).wait()
        pltpu.sync_copy(ones_v, cnt_sh.at[rowvB], add=True)

        @pl.when(i < NECH_ENC // 2 - 1)
        def _():
            _ldrow(2 * i + 3, rowvB, srB)
        return 0
    lax.fori_loop(0, NECH_ENC // 2, _deg, 0)

    plsc.subcore_barrier()

    pltpu.sync_copy(cnt_sh.at[pl.ds(s * RPS, RPS), :],
                    cnt_out.at[c, pl.ds(s * RPS, RPS), :])


_sc_encode_scratch = [
    pltpu.VMEM((72,), jnp.int32),
    pltpu.VMEM((72,), jnp.int32),
    pltpu.VMEM((72, D), jnp.float32),
    pltpu.VMEM((72, D), jnp.float32),
    pltpu.VMEM((8, D), jnp.float32),
    pltpu.VMEM((ECH_ENC,), jnp.int32),
    pltpu.VMEM((ECH_ENC,), jnp.int32),
    pltpu.VMEM((ECH_ENC, D), jnp.float32),
    pltpu.VMEM((80, D), jnp.float32),
    pltpu.VMEM_SHARED((NP, D), jnp.float32),
    pltpu.SemaphoreType.DMA,
    pltpu.SemaphoreType.DMA,
    pltpu.SemaphoreType.DMA,
    pltpu.SemaphoreType.DMA,
]

_sc_encode = pl.kernel(
    _sc_encode_body,
    out_type=(jax.ShapeDtypeStruct((N, D), jnp.float32),
              jax.ShapeDtypeStruct((NC, NP, D), jnp.float32)),
    mesh=_mesh,
    compiler_params=pltpu.CompilerParams(needs_layout_passes=False),
    scratch_types=_sc_encode_scratch,
)


def _sc_layer_body(hl_hbm, dinv80_hbm, ccr_hbm, T_hbm,
                   agg_out, ebufA, ebufB, colvA, rowvA, combovA,
                   colvB, rowvB, combovB,
                   gA, bA, mA, gB, bB, mB, dinv_v, tmp32, agg_sh,
                   sgA1, sgA2, ssA, sgB1, sgB2, ssB):
    c = lax.axis_index("c")
    s = lax.axis_index("s")
    wid = c * NS + s

    pltpu.sync_copy(dinv80_hbm, dinv_v)

    def _z(i, _):
        for d in range(8):
            mA[i, pl.ds(d * 16, 16)] = jnp.zeros((16,), jnp.float32)
        return 0
    lax.fori_loop(0, ECHUNK, _z, 0)
    for k in range(RPS // ECHUNK):
        pltpu.sync_copy(mA, agg_sh.at[pl.ds(s * RPS + k * ECHUNK, ECHUNK), :])
    plsc.subcore_barrier()

    def _load_e(cc, ebuf):
        base = (wid * NECHUNK + cc) * 3 * ECHUNK
        pltpu.sync_copy(ccr_hbm.at[pl.ds(base, 3 * ECHUNK)], ebuf)

    def _unpack_cc(ebuf, colv, combov):
        colv[pl.ds(0, 16)] = ebuf[pl.ds(0, 16)]
        colv[pl.ds(16, 16)] = ebuf[pl.ds(16, 16)]
        colv[pl.ds(24, 16)] = ebuf[pl.ds(24, 16)]
        combov[pl.ds(0, 16)] = ebuf[pl.ds(40, 16)]
        combov[pl.ds(16, 16)] = ebuf[pl.ds(56, 16)]
        combov[pl.ds(24, 16)] = ebuf[pl.ds(64, 16)]

    def _unpack_row(ebuf, rowv):
        rowv[pl.ds(0, 16)] = ebuf[pl.ds(80, 16)]
        rowv[pl.ds(16, 16)] = ebuf[pl.ds(96, 16)]
        rowv[pl.ds(24, 16)] = ebuf[pl.ds(104, 16)]

    def _start_g(colv, combov, g, b, s1, s2):
        pltpu.async_copy(hl_hbm.at[colv], g, s1)
        pltpu.async_copy(T_hbm.at[combov], b, s2)

    def _wait_g(colv, combov, g, b, s1, s2):
        pltpu.make_async_copy(hl_hbm.at[colv], g, s1).wait()
        pltpu.make_async_copy(T_hbm.at[combov], b, s2).wait()

    def _compute(colv, g, b, m):
        for base, jlo in ((0, 0), (16, 0), (24, 8)):
            cv = colv[pl.ds(base, 16)]
            dv = plsc.load_gather(
                dinv_v, [lax.shift_right_logical(cv, 7),
                         jnp.bitwise_and(cv, 127)])
            tmp32[pl.ds(16, 16)] = dv
            for j in range(jlo, 16):
                sv = plsc.load_gather(tmp32,
                                      [jnp.full((16,), 16 + j, jnp.int32)])
                jj = base + j
                for d in range(8):
                    u = (g[jj, pl.ds(d * 16, 16)]
                         + b[jj, pl.ds(d * 16, 16)]) * sv
                    m[jj, pl.ds(d * 16, 16)] = jnp.maximum(u, 0.0)

    _load_e(0, ebufA)
    _unpack_cc(ebufA, colvA, combovA)
    _unpack_row(ebufA, rowvA)
    _start_g(colvA, combovA, gA, bA, sgA1, sgA2)
    _load_e(1, ebufB)
    _unpack_cc(ebufB, colvB, combovB)
    _unpack_row(ebufB, rowvB)
    _start_g(colvB, combovB, gB, bB, sgB1, sgB2)

    def _iter(i, _):
        cA = 2 * i
        cB = 2 * i + 1

        @pl.when(i > 0)
        def _():
            pltpu.make_async_copy(mA, agg_sh.at[rowvA], ssA).wait()
            _unpack_row(ebufA, rowvA)
        _wait_g(colvA, combovA, gA, bA, sgA1, sgA2)
        _compute(colvA, gA, bA, mA)
        pltpu.async_copy(mA, agg_sh.at[rowvA], ssA, add=True)

        @pl.when(i < NPAIR - 1)
        def _():
            _load_e(cA + 2, ebufA)
            _unpack_cc(ebufA, colvA, combovA)
            _start_g(colvA, combovA, gA, bA, sgA1, sgA2)

        @pl.when(i > 0)
        def _():
            pltpu.make_async_copy(mB, agg_sh.at[rowvB], ssB).wait()
            _unpack_row(ebufB, rowvB)
        _wait_g(colvB, combovB, gB, bB, sgB1, sgB2)
        _compute(colvB, gB, bB, mB)
        pltpu.async_copy(mB, agg_sh.at[rowvB], ssB, add=True)

        @pl.when(i < NPAIR - 1)
        def _():
            _load_e(cB + 2, ebufB)
            _unpack_cc(ebufB, colvB, combovB)
            _start_g(colvB, combovB, gB, bB, sgB1, sgB2)
        return 0
    lax.fori_loop(0, NPAIR, _iter, 0)
    pltpu.make_async_copy(mA, agg_sh.at[rowvA], ssA).wait()
    pltpu.make_async_copy(mB, agg_sh.at[rowvB], ssB).wait()
    plsc.subcore_barrier()

    pltpu.sync_copy(agg_sh.at[pl.ds(s * RPS, RPS), :],
                    agg_out.at[c, pl.ds(s * RPS, RPS), :])


_sc_layer_scratch = [
    pltpu.VMEM((3 * ECHUNK,), jnp.int32),
    pltpu.VMEM((3 * ECHUNK,), jnp.int32),
    pltpu.VMEM((ECHUNK,), jnp.int32),
    pltpu.VMEM((ECHUNK,), jnp.int32),
    pltpu.VMEM((ECHUNK,), jnp.int32),
    pltpu.VMEM((ECHUNK,), jnp.int32),
    pltpu.VMEM((ECHUNK,), jnp.int32),
    pltpu.VMEM((ECHUNK,), jnp.int32),
    pltpu.VMEM((ECHUNK, D), jnp.float32),
    pltpu.VMEM((ECHUNK, D), jnp.float32),
    pltpu.VMEM((ECHUNK, D), jnp.float32),
    pltpu.VMEM((ECHUNK, D), jnp.float32),
    pltpu.VMEM((ECHUNK, D), jnp.float32),
    pltpu.VMEM((ECHUNK, D), jnp.float32),
    pltpu.VMEM((80, D), jnp.float32),
    pltpu.VMEM((32,), jnp.float32),
    pltpu.VMEM_SHARED((NP, D), jnp.float32),
    pltpu.SemaphoreType.DMA,
    pltpu.SemaphoreType.DMA,
    pltpu.SemaphoreType.DMA,
    pltpu.SemaphoreType.DMA,
    pltpu.SemaphoreType.DMA,
    pltpu.SemaphoreType.DMA,
]

_sc_layer = pl.kernel(
    _sc_layer_body,
    out_type=jax.ShapeDtypeStruct((NC, NP, D), jnp.float32),
    mesh=_mesh,
    compiler_params=pltpu.CompilerParams(needs_layout_passes=False),
    scratch_types=_sc_layer_scratch,
)


def _tc_prep_kernel(cnt_ref, h0_ref, w_ref, b_ref, hl_ref, dinv_ref):
    cnt = cnt_ref[0, :, 0:1] + cnt_ref[1, :, 0:1]
    dinv_ref[...] = lax.rsqrt(cnt + 1.0)
    hl_ref[...] = jnp.dot(h0_ref[...], w_ref[...],
                          preferred_element_type=jnp.float32) + b_ref[...]


_tc_prep = pl.pallas_call(
    _tc_prep_kernel,
    out_shape=(jax.ShapeDtypeStruct((N, D), jnp.float32),
               jax.ShapeDtypeStruct((NP, 1), jnp.float32)),
)


def _tc_layer_kernel(agg_ref, hl_ref, dinv_ref, root_ref, g_ref, bt_ref,
                     w_ref, b_ref, out_ref):
    dinv = dinv_ref[:N, :]
    hl = hl_ref[...]
    u = agg_ref[0, :N, :] + agg_ref[1, :N, :]
    out = dinv * u + jnp.maximum(hl + root_ref[...], 0.0) * (dinv * dinv)
    mu = jnp.mean(out, axis=0, keepdims=True)
    var = jnp.mean((out - mu) ** 2, axis=0, keepdims=True)
    out = (out - mu) / jnp.sqrt(var + 1e-5) * g_ref[...] + bt_ref[...]
    out = jnp.maximum(out, 0.0)
    out_ref[...] = jnp.dot(out, w_ref[...],
                           preferred_element_type=jnp.float32) + b_ref[...]


_tc_layer = pl.pallas_call(
    _tc_layer_kernel,
    out_shape=jax.ShapeDtypeStruct((N, D), jnp.float32),
)


def _tc_final_kernel(agg_ref, hl_ref, dinv_ref, root_ref, g_ref, bt_ref,
                     batch_ref, pw_ref, pb_ref, out_ref):
    dinv = dinv_ref[:N, :]
    hl = hl_ref[...]
    u = agg_ref[0, :N, :] + agg_ref[1, :N, :]
    out = dinv * u + jnp.maximum(hl + root_ref[...], 0.0) * (dinv * dinv)
    mu = jnp.mean(out, axis=0, keepdims=True)
    var = jnp.mean((out - mu) ** 2, axis=0, keepdims=True)
    out = (out - mu) / jnp.sqrt(var + 1e-5) * g_ref[...] + bt_ref[...]
    gid = lax.broadcasted_iota(jnp.int32, (N, NGRAPHS), 1)
    oh = jnp.where(batch_ref[...] == gid, 1.0, 0.0).astype(jnp.float32)
    hg = lax.dot_general(oh, out, (((0,), (0,)), ((), ())),
                         preferred_element_type=jnp.float32)
    out_ref[...] = jnp.dot(hg, pw_ref[...],
                           preferred_element_type=jnp.float32) + pb_ref[...]


_tc_final = pl.pallas_call(
    _tc_final_kernel,
    out_shape=jax.ShapeDtypeStruct((NGRAPHS, NGRAPHS), jnp.float32),
)


def kernel(x, edge_index, edge_attr, batch, atom_tab, lin_W, lin_b, root,
           bond_tab, bn_gamma, bn_beta, pred_W, pred_b):
    i32 = jnp.int32
    row = edge_index[0].astype(i32)
    col = edge_index[1].astype(i32)
    combo = (edge_attr[:, 0] * 64 + edge_attr[:, 1] * 8
             + edge_attr[:, 2]).astype(i32)
    aidx = (x.astype(i32) + 64 * jnp.arange(9, dtype=i32)[None, :]).reshape(-1)
    atom_flat = atom_tab.reshape(9 * 64, D)
    T = (bond_tab[:, 0][:, :, None, None, :]
         + bond_tab[:, 1][:, None, :, None, :]
         + bond_tab[:, 2][:, None, None, :, :]).reshape(NLAYERS, 512, D)

    ccr = jnp.stack([col.reshape(-1, ECHUNK), combo.reshape(-1, ECHUNK),
                     row.reshape(-1, ECHUNK)], axis=1).reshape(-1)
    h0, cnt = _sc_encode(aidx, row, atom_flat)
    hl, dinvc = _tc_prep(cnt, h0, lin_W[0], lin_b[0].reshape(1, D))
    dinv80 = dinvc.reshape(NP // D, D)
    for l in range(NLAYERS):
        aggP = _sc_layer(hl, dinv80, ccr, T[l])
        if l < NLAYERS - 1:
            hl = _tc_layer(aggP, hl, dinvc, root[l].reshape(1, D),
                           bn_gamma[l].reshape(1, D), bn_beta[l].reshape(1, D),
                           lin_W[l + 1], lin_b[l + 1].reshape(1, D))
        else:
            out = _tc_final(aggP, hl, dinvc, root[l].reshape(1, D),
                            bn_gamma[l].reshape(1, D), bn_beta[l].reshape(1, D),
                            batch.astype(i32).reshape(N, 1), pred_W,
                            pred_b.reshape(1, NGRAPHS))
    return out


# speedup vs baseline: 11.9828x; 1.0231x over previous
import jax
import jax.numpy as jnp
from jax import lax
from jax.experimental import pallas as pl
from jax.experimental.pallas import tpu as pltpu
from jax.experimental.pallas import tpu_sc as plsc

N = 10000
E = 320000
D = 128
NLAYERS = 5
NGRAPHS = 128

NC = 2
NS = 16
NW = NC * NS

EPT = E // NW
ECHUNK = 40
NECHUNK = EPT // ECHUNK
NPAIR = NECHUNK // 2
ECH_ENC = 40
NECH_ENC = EPT // ECH_ENC
NPT = 320
NP = 10240
RPS = NP // NS

_mesh = plsc.VectorSubcoreMesh(core_axis_name="c", subcore_axis_name="s",
                               num_cores=NC, num_subcores=NS)


def _sc_encode_body(aidx_hbm, row_hbm, atom_hbm, h0_out, cnt_out,
                    idxA, idxB, gbufA, gbufB, hbuf, rowvA, rowvB, ones_v,
                    zbuf, cnt_sh, semA, semB, srA, srB):
    c = lax.axis_index("c")
    s = lax.axis_index("s")
    wid = c * NS + s

    def _z(i, _):
        for d in range(8):
            zbuf[i, pl.ds(d * 16, 16)] = jnp.zeros((16,), jnp.float32)
        return 0
    lax.fori_loop(0, 80, _z, 0)
    for k in range(8):
        pltpu.sync_copy(zbuf, cnt_sh.at[pl.ds(s * RPS + k * 80, 80), :])

    def _o(i, _):
        for d in range(8):
            ones_v[i, pl.ds(d * 16, 16)] = jnp.ones((16,), jnp.float32)
        return 0
    lax.fori_loop(0, ECH_ENC, _o, 0)
    plsc.subcore_barrier()

    nnodes = jnp.maximum(0, jnp.minimum(NPT, N - wid * NPT))
    nch = nnodes // 8

    def _prefetch(i, idxv, gbuf, sem):
        nbase = wid * NPT + i * 8
        pltpu.sync_copy(aidx_hbm.at[pl.ds(nbase * 9, 72)], idxv)
        pltpu.async_copy(atom_hbm.at[idxv], gbuf, sem)

    def _consume(i, idxv, gbuf, sem):
        nbase = wid * NPT + i * 8
        pltpu.make_async_copy(atom_hbm.at[idxv], gbuf, sem).wait()
        for j in range(8):
            for d in range(8):
                acc = gbuf[j * 9, pl.ds(d * 16, 16)]
                for k in range(1, 9):
                    acc = acc + gbuf[j * 9 + k, pl.ds(d * 16, 16)]
                hbuf[j, pl.ds(d * 16, 16)] = acc
        pltpu.sync_copy(hbuf, h0_out.at[pl.ds(nbase, 8), :])

    _prefetch(0, idxA, gbufA, semA)
    _prefetch(1, idxB, gbufB, semB)

    def _pair(i, _):
        _consume(2 * i, idxA, gbufA, semA)

        @pl.when(2 * i + 2 < nch)
        def _():
            _prefetch(2 * i + 2, idxA, gbufA, semA)
        _consume(2 * i + 1, idxB, gbufB, semB)

        @pl.when(2 * i + 3 < nch)
        def _():
            _prefetch(2 * i + 3, idxB, gbufB, semB)
        return 0
    lax.fori_loop(0, nch // 2, _pair, 0)

    def _ldrow(i, rowv, sem):
        ebase = wid * EPT + i * ECH_ENC
        pltpu.async_copy(row_hbm.at[pl.ds(ebase, ECH_ENC)], rowv, sem)

    _ldrow(0, rowvA, srA)
    _ldrow(1, rowvB, srB)

    def _deg(i, _):
        pltpu.make_async_copy(row_hbm.at[pl.ds(0, ECH_ENC)], rowvA, srA).wait()
        pltpu.sync_copy(ones_v, cnt_sh.at[rowvA], add=True)

        @pl.when(i < NECH_ENC // 2 - 1)
        def _():
            _ldrow(2 * i + 2, rowvA, srA)
        pltpu.make_async_copy(row_hbm.at[pl.ds(0, ECH_ENC)], rowvB, srB).wait()
        pltpu.sync_copy(ones_v, cnt_sh.at[rowvB], add=True)

        @pl.when(i < NECH_ENC // 2 - 1)
        def _():
            _ldrow(2 * i + 3, rowvB, srB)
        return 0
    lax.fori_loop(0, NECH_ENC // 2, _deg, 0)

    plsc.subcore_barrier()

    pltpu.sync_copy(cnt_sh.at[pl.ds(s * RPS, RPS), :],
                    cnt_out.at[c, pl.ds(s * RPS, RPS), :])


_sc_encode_scratch = [
    pltpu.VMEM((72,), jnp.int32),
    pltpu.VMEM((72,), jnp.int32),
    pltpu.VMEM((72, D), jnp.float32),
    pltpu.VMEM((72, D), jnp.float32),
    pltpu.VMEM((8, D), jnp.float32),
    pltpu.VMEM((ECH_ENC,), jnp.int32),
    pltpu.VMEM((ECH_ENC,), jnp.int32),
    pltpu.VMEM((ECH_ENC, D), jnp.float32),
    pltpu.VMEM((80, D), jnp.float32),
    pltpu.VMEM_SHARED((NP, D), jnp.float32),
    pltpu.SemaphoreType.DMA,
    pltpu.SemaphoreType.DMA,
    pltpu.SemaphoreType.DMA,
    pltpu.SemaphoreType.DMA,
]

_sc_encode = pl.kernel(
    _sc_encode_body,
    out_type=(jax.ShapeDtypeStruct((N, D), jnp.float32),
              jax.ShapeDtypeStruct((NC, NP, D), jnp.float32)),
    mesh=_mesh,
    compiler_params=pltpu.CompilerParams(needs_layout_passes=False),
    scratch_types=_sc_encode_scratch,
)


def _sc_layer_body(hl_hbm, dinv80_hbm, ccr_hbm, T_hbm,
                   agg_out, ebufP, colvA, rowvA, combovA,
                   colvB, rowvB, combovB,
                   gA, bA, mA, gB, bB, mB, dinv_v, tmp32, agg_sh,
                   sgA1, sgA2, ssA, sgB1, sgB2, ssB):
    c = lax.axis_index("c")
    s = lax.axis_index("s")
    wid = c * NS + s

    pltpu.sync_copy(dinv80_hbm, dinv_v)

    def _z(i, _):
        for d in range(8):
            mA[i, pl.ds(d * 16, 16)] = jnp.zeros((16,), jnp.float32)
        return 0
    lax.fori_loop(0, ECHUNK, _z, 0)
    for k in range(RPS // ECHUNK):
        pltpu.sync_copy(mA, agg_sh.at[pl.ds(s * RPS + k * ECHUNK, ECHUNK), :])
    plsc.subcore_barrier()

    def _load_pair(i, ebuf):
        base = (wid * NECHUNK + 2 * i) * 3 * ECHUNK
        pltpu.sync_copy(ccr_hbm.at[pl.ds(base, 6 * ECHUNK)], ebuf)

    def _unpack_cc(ebuf, off, colv, combov):
        colv[pl.ds(0, 16)] = ebuf[pl.ds(off + 0, 16)]
        colv[pl.ds(16, 16)] = ebuf[pl.ds(off + 16, 16)]
        colv[pl.ds(24, 16)] = ebuf[pl.ds(off + 24, 16)]
        combov[pl.ds(0, 16)] = ebuf[pl.ds(off + 40, 16)]
        combov[pl.ds(16, 16)] = ebuf[pl.ds(off + 56, 16)]
        combov[pl.ds(24, 16)] = ebuf[pl.ds(off + 64, 16)]

    def _unpack_row(ebuf, off, rowv):
        rowv[pl.ds(0, 16)] = ebuf[pl.ds(off + 80, 16)]
        rowv[pl.ds(16, 16)] = ebuf[pl.ds(off + 96, 16)]
        rowv[pl.ds(24, 16)] = ebuf[pl.ds(off + 104, 16)]

    def _start_g(colv, combov, g, b, s1, s2):
        pltpu.async_copy(hl_hbm.at[colv], g, s1)
        pltpu.async_copy(T_hbm.at[combov], b, s2)

    def _wait_g(colv, combov, g, b, s1, s2):
        pltpu.make_async_copy(hl_hbm.at[colv], g, s1).wait()
        pltpu.make_async_copy(T_hbm.at[combov], b, s2).wait()

    def _compute(colv, g, b, m):
        for base, jlo in ((0, 0), (16, 0), (24, 8)):
            cv = colv[pl.ds(base, 16)]
            dv = plsc.load_gather(
                dinv_v, [lax.shift_right_logical(cv, 7),
                         jnp.bitwise_and(cv, 127)])
            tmp32[pl.ds(16, 16)] = dv
            for j in range(jlo, 16):
                sv = plsc.load_gather(tmp32,
                                      [jnp.full((16,), 16 + j, jnp.int32)])
                jj = base + j
                for d in range(8):
                    u = (g[jj, pl.ds(d * 16, 16)]
                         + b[jj, pl.ds(d * 16, 16)]) * sv
                    m[jj, pl.ds(d * 16, 16)] = jnp.maximum(u, 0.0)

    _load_pair(0, ebufP)
    _unpack_cc(ebufP, 0, colvA, combovA)
    _unpack_row(ebufP, 0, rowvA)
    _start_g(colvA, combovA, gA, bA, sgA1, sgA2)
    _unpack_cc(ebufP, 120, colvB, combovB)
    _unpack_row(ebufP, 120, rowvB)
    _start_g(colvB, combovB, gB, bB, sgB1, sgB2)

    def _iter(i, _):
        @pl.when(i > 0)
        def _():
            pltpu.make_async_copy(mA, agg_sh.at[rowvA], ssA).wait()
            _unpack_row(ebufP, 0, rowvA)
            pltpu.make_async_copy(mB, agg_sh.at[rowvB], ssB).wait()
            _unpack_row(ebufP, 120, rowvB)
        _wait_g(colvA, combovA, gA, bA, sgA1, sgA2)
        _compute(colvA, gA, bA, mA)
        pltpu.async_copy(mA, agg_sh.at[rowvA], ssA, add=True)

        @pl.when(i < NPAIR - 1)
        def _():
            _load_pair(i + 1, ebufP)
            _unpack_cc(ebufP, 0, colvA, combovA)
            _start_g(colvA, combovA, gA, bA, sgA1, sgA2)
        _wait_g(colvB, combovB, gB, bB, sgB1, sgB2)
        _compute(colvB, gB, bB, mB)
        pltpu.async_copy(mB, agg_sh.at[rowvB], ssB, add=True)

        @pl.when(i < NPAIR - 1)
        def _():
            _unpack_cc(ebufP, 120, colvB, combovB)
            _start_g(colvB, combovB, gB, bB, sgB1, sgB2)
        return 0
    lax.fori_loop(0, NPAIR, _iter, 0)
    pltpu.make_async_copy(mA, agg_sh.at[rowvA], ssA).wait()
    pltpu.make_async_copy(mB, agg_sh.at[rowvB], ssB).wait()
    plsc.subcore_barrier()

    pltpu.sync_copy(agg_sh.at[pl.ds(s * RPS, RPS), :],
                    agg_out.at[c, pl.ds(s * RPS, RPS), :])


_sc_layer_scratch = [
    pltpu.VMEM((6 * ECHUNK,), jnp.int32),
    pltpu.VMEM((ECHUNK,), jnp.int32),
    pltpu.VMEM((ECHUNK,), jnp.int32),
    pltpu.VMEM((ECHUNK,), jnp.int32),
    pltpu.VMEM((ECHUNK,), jnp.int32),
    pltpu.VMEM((ECHUNK,), jnp.int32),
    pltpu.VMEM((ECHUNK,), jnp.int32),
    pltpu.VMEM((ECHUNK, D), jnp.float32),
    pltpu.VMEM((ECHUNK, D), jnp.float32),
    pltpu.VMEM((ECHUNK, D), jnp.float32),
    pltpu.VMEM((ECHUNK, D), jnp.float32),
    pltpu.VMEM((ECHUNK, D), jnp.float32),
    pltpu.VMEM((ECHUNK, D), jnp.float32),
    pltpu.VMEM((80, D), jnp.float32),
    pltpu.VMEM((32,), jnp.float32),
    pltpu.VMEM_SHARED((NP, D), jnp.float32),
    pltpu.SemaphoreType.DMA,
    pltpu.SemaphoreType.DMA,
    pltpu.SemaphoreType.DMA,
    pltpu.SemaphoreType.DMA,
    pltpu.SemaphoreType.DMA,
    pltpu.SemaphoreType.DMA,
]

_sc_layer = pl.kernel(
    _sc_layer_body,
    out_type=jax.ShapeDtypeStruct((NC, NP, D), jnp.float32),
    mesh=_mesh,
    compiler_params=pltpu.CompilerParams(needs_layout_passes=False),
    scratch_types=_sc_layer_scratch,
)


def _tc_prep_kernel(cnt_ref, h0_ref, w_ref, b_ref, hl_ref, dinv_ref):
    cnt = cnt_ref[0, :, 0:1] + cnt_ref[1, :, 0:1]
    dinv_ref[...] = lax.rsqrt(cnt + 1.0)
    hl_ref[...] = jnp.dot(h0_ref[...], w_ref[...],
                          preferred_element_type=jnp.float32) + b_ref[...]


_tc_prep = pl.pallas_call(
    _tc_prep_kernel,
    out_shape=(jax.ShapeDtypeStruct((N, D), jnp.float32),
               jax.ShapeDtypeStruct((NP, 1), jnp.float32)),
)


def _tc_layer_kernel(agg_ref, hl_ref, dinv_ref, root_ref, g_ref, bt_ref,
                     w_ref, b_ref, out_ref):
    dinv = dinv_ref[:N, :]
    hl = hl_ref[...]
    u = agg_ref[0, :N, :] + agg_ref[1, :N, :]
    out = dinv * u + jnp.maximum(hl + root_ref[...], 0.0) * (dinv * dinv)
    mu = jnp.mean(out, axis=0, keepdims=True)
    var = jnp.mean((out - mu) ** 2, axis=0, keepdims=True)
    out = (out - mu) / jnp.sqrt(var + 1e-5) * g_ref[...] + bt_ref[...]
    out = jnp.maximum(out, 0.0)
    out_ref[...] = jnp.dot(out, w_ref[...],
                           preferred_element_type=jnp.float32) + b_ref[...]


_tc_layer = pl.pallas_call(
    _tc_layer_kernel,
    out_shape=jax.ShapeDtypeStruct((N, D), jnp.float32),
)


def _tc_final_kernel(agg_ref, hl_ref, dinv_ref, root_ref, g_ref, bt_ref,
                     batch_ref, pw_ref, pb_ref, out_ref):
    dinv = dinv_ref[:N, :]
    hl = hl_ref[...]
    u = agg_ref[0, :N, :] + agg_ref[1, :N, :]
    out = dinv * u + jnp.maximum(hl + root_ref[...], 0.0) * (dinv * dinv)
    mu = jnp.mean(out, axis=0, keepdims=True)
    var = jnp.mean((out - mu) ** 2, axis=0, keepdims=True)
    out = (out - mu) / jnp.sqrt(var + 1e-5) * g_ref[...] + bt_ref[...]
    gid = lax.broadcasted_iota(jnp.int32, (N, NGRAPHS), 1)
    oh = jnp.where(batch_ref[...] == gid, 1.0, 0.0).astype(jnp.float32)
    hg = lax.dot_general(oh, out, (((0,), (0,)), ((), ())),
                         preferred_element_type=jnp.float32)
    out_ref[...] = jnp.dot(hg, pw_ref[...],
                           preferred_element_type=jnp.float32) + pb_ref[...]


_tc_final = pl.pallas_call(
    _tc_final_kernel,
    out_shape=jax.ShapeDtypeStruct((NGRAPHS, NGRAPHS), jnp.float32),
)


def kernel(x, edge_index, edge_attr, batch, atom_tab, lin_W, lin_b, root,
           bond_tab, bn_gamma, bn_beta, pred_W, pred_b):
    i32 = jnp.int32
    row = edge_index[0].astype(i32)
    col = edge_index[1].astype(i32)
    combo = (edge_attr[:, 0] * 64 + edge_attr[:, 1] * 8
             + edge_attr[:, 2]).astype(i32)
    aidx = (x.astype(i32) + 64 * jnp.arange(9, dtype=i32)[None, :]).reshape(-1)
    atom_flat = atom_tab.reshape(9 * 64, D)
    T = (bond_tab[:, 0][:, :, None, None, :]
         + bond_tab[:, 1][:, None, :, None, :]
         + bond_tab[:, 2][:, None, None, :, :]).reshape(NLAYERS, 512, D)

    ccr = jnp.stack([col.reshape(-1, ECHUNK), combo.reshape(-1, ECHUNK),
                     row.reshape(-1, ECHUNK)], axis=1).reshape(-1)
    h0, cnt = _sc_encode(aidx, row, atom_flat)
    hl, dinvc = _tc_prep(cnt, h0, lin_W[0], lin_b[0].reshape(1, D))
    dinv80 = dinvc.reshape(NP // D, D)
    for l in range(NLAYERS):
        aggP = _sc_layer(hl, dinv80, ccr, T[l])
        if l < NLAYERS - 1:
            hl = _tc_layer(aggP, hl, dinvc, root[l].reshape(1, D),
                           bn_gamma[l].reshape(1, D), bn_beta[l].reshape(1, D),
                           lin_W[l + 1], lin_b[l + 1].reshape(1, D))
        else:
            out = _tc_final(aggP, hl, dinvc, root[l].reshape(1, D),
                            bn_gamma[l].reshape(1, D), bn_beta[l].reshape(1, D),
                            batch.astype(i32).reshape(N, 1), pred_W,
                            pred_b.reshape(1, NGRAPHS))
    return out
